# baseline (speedup 1.0000x reference)
import functools

import jax
import jax.numpy as jnp
from jax import lax
from jax.experimental import pallas as pl
from jax.experimental.pallas import tpu as pltpu
from jax.experimental.pallas import tpu_sc as plsc

BATCH = 4096
SEQ = 200
EMBED = 64
NUM_WORKERS = 32
BLK = BATCH // NUM_WORKERS
NSLOT = 4
CT = EMBED // 8
LANES = 16

_mesh = plsc.VectorSubcoreMesh(core_axis_name="c", subcore_axis_name="s")


@functools.partial(
    pl.kernel,
    out_type=jax.ShapeDtypeStruct((SEQ, CT, NUM_WORKERS, 8, BLK), jnp.float32),
    mesh=_mesh,
    scratch_types=[
        pltpu.VMEM((SEQ, BLK), jnp.int32),
        pltpu.VMEM((SEQ, EMBED), jnp.float32),
        pltpu.VMEM((NSLOT, BLK, EMBED), jnp.float32),
        pltpu.VMEM((NSLOT, CT, 8, BLK), jnp.float32),
        pltpu.SemaphoreType.DMA((NSLOT,)),
        pltpu.SemaphoreType.DMA((NSLOT,)),
    ],
    compiler_params=pltpu.CompilerParams(
        use_tc_tiling_on_sc=False, needs_layout_passes=False
    ),
)
def _embed_kernel(x_hbm, table_hbm, pe_hbm, out_hbm,
                  idx_v, pe_v, rows_v, obuf_v, gsem, osem):
    wid = lax.axis_index("s") * 2 + lax.axis_index("c")
    pltpu.sync_copy(pe_hbm, pe_v)
    pltpu.sync_copy(x_hbm.at[wid], idx_v)

    lane = jnp.arange(LANES, dtype=jnp.int32)

    def fire_gather(s, slot):
        pltpu.async_copy(table_hbm.at[idx_v.at[s]], rows_v.at[slot],
                         gsem.at[slot])

    fire_gather(0, 0)
    fire_gather(1, 1)

    def process(s, slot):
        nslot = (slot + 2) % NSLOT

        @pl.when(s + 2 < SEQ)
        def _():
            @pl.when(s >= 2)
            def _():
                for ct in range(CT):
                    pltpu.make_async_copy(
                        obuf_v.at[nslot, ct],
                        out_hbm.at[0, 0, 0],
                        osem.at[nslot],
                    ).wait()
            fire_gather(s + 2, nslot)

        pltpu.make_async_copy(
            table_hbm.at[pl.ds(0, BLK)],
            rows_v.at[slot],
            gsem.at[slot],
        ).wait()

        @plsc.parallel_loop(0, EMBED, unroll=2)
        def col_body(c):
            ct = c // 8
            cm = lax.rem(c, 8)
            cvec = jnp.full((LANES,), c, dtype=jnp.int32)
            pe_b = plsc.load_gather(
                pe_v, [jnp.full((LANES,), s, dtype=jnp.int32), cvec])
            for bv in range(BLK // LANES):
                rows16 = plsc.load_gather(
                    rows_v.at[slot], [lane + bv * LANES, cvec])
                obuf_v[slot, ct, cm, pl.ds(bv * LANES, LANES)] = rows16 + pe_b

        for ct in range(CT):
            pltpu.async_copy(
                obuf_v.at[slot, ct],
                out_hbm.at[s, ct, wid],
                osem.at[slot],
            )

    def quad_body(q, carry):
        for bb in range(NSLOT):
            process(q * NSLOT + bb, bb)
        return carry

    lax.fori_loop(0, SEQ // NSLOT, quad_body, 0)

    for slot in range(NSLOT):
        for ct in range(CT):
            pltpu.make_async_copy(
                obuf_v.at[slot, ct],
                out_hbm.at[0, 0, 0],
                osem.at[slot],
            ).wait()


def kernel(x, table, pe):
    x_arr = x.T.reshape(SEQ, NUM_WORKERS, BLK).transpose(1, 0, 2)
    pe_s = pe[0, :SEQ, :]
    out6 = _embed_kernel(x_arr, table, pe_s)
    return out6.transpose(2, 4, 0, 1, 3).reshape(BATCH, SEQ, EMBED)


# speedup vs baseline: 1.7352x; 1.7352x over previous
import functools

import jax
import jax.numpy as jnp
from jax import lax
from jax.experimental import pallas as pl
from jax.experimental.pallas import tpu as pltpu
from jax.experimental.pallas import tpu_sc as plsc

BATCH = 4096
SEQ = 200
EMBED = 64
NUM_WORKERS = 32
BLK = BATCH // NUM_WORKERS
NSLOT = 4
CT = EMBED // 8
LANES = 16

_mesh = plsc.VectorSubcoreMesh(core_axis_name="c", subcore_axis_name="s")


@functools.partial(
    pl.kernel,
    out_type=jax.ShapeDtypeStruct((SEQ, CT, NUM_WORKERS, 8, BLK), jnp.float32),
    mesh=_mesh,
    scratch_types=[
        pltpu.VMEM((SEQ, BLK), jnp.int32),
        pltpu.VMEM((SEQ, EMBED), jnp.float32),
        pltpu.VMEM((NSLOT, BLK, EMBED), jnp.float32),
        pltpu.VMEM((NSLOT, CT, 8, BLK), jnp.float32),
        pltpu.SemaphoreType.DMA((NSLOT,)),
        pltpu.SemaphoreType.DMA((NSLOT,)),
    ],
    compiler_params=pltpu.CompilerParams(
        use_tc_tiling_on_sc=False, needs_layout_passes=False
    ),
)
def _embed_kernel(x_hbm, table_hbm, pe_hbm, out_hbm,
                  idx_v, pe_v, rows_v, obuf_v, gsem, osem):
    wid = lax.axis_index("s") * 2 + lax.axis_index("c")
    pltpu.sync_copy(pe_hbm, pe_v)
    pltpu.sync_copy(x_hbm.at[wid], idx_v)

    lane = jnp.arange(LANES, dtype=jnp.int32)

    def fire_gather(s, slot):
        pltpu.async_copy(table_hbm.at[idx_v.at[s]], rows_v.at[slot],
                         gsem.at[slot])

    fire_gather(0, 0)
    fire_gather(1, 1)
    fire_gather(2, 2)

    def process(s, slot):
        @pl.when(s >= NSLOT)
        def _():
            for ct in range(CT):
                pltpu.make_async_copy(
                    obuf_v.at[slot, ct],
                    out_hbm.at[0, 0, 0],
                    osem.at[slot],
                ).wait()

        @pl.when(s + 3 < SEQ)
        def _():
            fire_gather(s + 3, (slot + 3) % NSLOT)

        pltpu.make_async_copy(
            table_hbm.at[pl.ds(0, BLK)],
            rows_v.at[slot],
            gsem.at[slot],
        ).wait()

        rows_ref = rows_v.at[slot]
        obuf_ref = obuf_v.at[slot]
        svec = jnp.full((LANES,), s, dtype=jnp.int32)

        @plsc.parallel_loop(0, EMBED, unroll=2)
        def col_body(i):
            cvec = (i >> 4) * LANES + ((lane + (i & (LANES - 1)))
                                       & (LANES - 1))
            ctv = cvec >> 3
            cmv = cvec & 7
            pe_b = plsc.load_gather(pe_v, [svec, cvec])
            for bv in range(BLK // LANES):
                bmv = lane + bv * LANES
                rows16 = plsc.load_gather(rows_ref, [bmv, cvec])
                plsc.store_scatter(obuf_ref, [ctv, cmv, bmv],
                                   rows16 + pe_b)

        for ct in range(CT):
            pltpu.async_copy(
                obuf_ref.at[ct],
                out_hbm.at[s, ct, wid],
                osem.at[slot],
            )

    def quad_body(q, carry):
        for bb in range(NSLOT):
            process(q * NSLOT + bb, bb)
        return carry

    lax.fori_loop(0, SEQ // NSLOT, quad_body, 0)

    for slot in range(NSLOT):
        for ct in range(CT):
            pltpu.make_async_copy(
                obuf_v.at[slot, ct],
                out_hbm.at[0, 0, 0],
                osem.at[slot],
            ).wait()


def kernel(x, table, pe):
    x_arr = x.T.reshape(SEQ, NUM_WORKERS, BLK).transpose(1, 0, 2)
    pe_s = pe[0, :SEQ, :]
    out6 = _embed_kernel(x_arr, table, pe_s)
    return out6.transpose(2, 4, 0, 1, 3).reshape(BATCH, SEQ, EMBED)
